# baseline (speedup 1.0000x reference)
import functools

import jax
import jax.numpy as jnp
from jax import lax
from jax.experimental import pallas as pl
from jax.experimental.pallas import tpu as pltpu
from jax.experimental.pallas import tpu_sc as plsc

B = 16384
D = 64
NC = 1
NS = 16
NW = NC * NS
ROWS_PER_W = B // NW
CHUNK = 128
NCHUNK = ROWS_PER_W // CHUNK

_L = 16
_ND = D // _L

_DNUMS = lax.GatherDimensionNumbers(
    offset_dims=(), collapsed_slice_dims=(0,), start_index_map=(0,))


def _shuffle(v, perm):
    return lax.gather(v, perm, _DNUMS, slice_sizes=(1,),
                      mode=lax.GatherScatterMode.PROMISE_IN_BOUNDS)


def _sum16(v):
    lanes = lax.iota(jnp.int32, _L)
    for k in (1, 2, 4, 8):
        perm = jnp.reshape(lanes ^ k, (_L, 1))
        v = v + _shuffle(v, perm)
    return v


def _rsqrt_guard(s):
    i = lax.bitcast_convert_type(s, jnp.int32)
    i = jnp.int32(0x5F3759DF) - lax.shift_right_logical(i, 1)
    y = lax.bitcast_convert_type(i, jnp.float32)
    half = s * jnp.float32(0.5)
    for _ in range(3):
        y = y * (jnp.float32(1.5) - half * y * y)
    return jnp.minimum(y, jnp.float32(1e12))


def _stage_body(h_idx_hbm, t_idx_hbm, ent_hbm, h2_hbm, t2_hbm,
                hidx_v, tidx_v, hp_v, tp_v, h_v, t_v, sem):
    wid = lax.axis_index("s") * NC + lax.axis_index("c")

    def chunk_body(c, carry):
        base = wid * ROWS_PER_W + c * CHUNK
        pltpu.sync_copy(h_idx_hbm.at[pl.ds(base, CHUNK)], hidx_v)
        pltpu.sync_copy(t_idx_hbm.at[pl.ds(base, CHUNK)], tidx_v)

        def pair_body(g, carry2):
            s = pl.ds(g * _L, _L)
            hp_v[s] = lax.shift_right_logical(hidx_v[s], 1)
            tp_v[s] = lax.shift_right_logical(tidx_v[s], 1)
            return carry2

        lax.fori_loop(0, CHUNK // _L, pair_body, 0, unroll=False)
        d1 = pltpu.async_copy(ent_hbm.at[hp_v], h_v, sem)
        d2 = pltpu.async_copy(ent_hbm.at[tp_v], t_v, sem)
        d1.wait()
        d2.wait()
        pltpu.sync_copy(h_v, h2_hbm.at[pl.ds(base, CHUNK)])
        pltpu.sync_copy(t_v, t2_hbm.at[pl.ds(base, CHUNK)])
        return carry

    lax.fori_loop(0, NCHUNK, chunk_body, 0, unroll=False)


def _math_body(h_idx_hbm, r_idx_hbm, t_idx_hbm, rel_hbm, nrm_hbm,
               h2_hbm, t2_hbm, out_hbm, hidx_v, ridx_v, tidx_v, rp_v,
               h_v, r_v, t_v, n_v, out_v, sem):
    wid = lax.axis_index("s") * NC + lax.axis_index("c")

    def chunk_body(c, carry):
        base = wid * ROWS_PER_W + c * CHUNK
        pltpu.sync_copy(h_idx_hbm.at[pl.ds(base, CHUNK)], hidx_v)
        pltpu.sync_copy(r_idx_hbm.at[pl.ds(base, CHUNK)], ridx_v)
        pltpu.sync_copy(t_idx_hbm.at[pl.ds(base, CHUNK)], tidx_v)

        def pair_body(g, carry2):
            s = pl.ds(g * _L, _L)
            rp_v[s] = lax.shift_right_logical(ridx_v[s], 1)
            return carry2

        lax.fori_loop(0, CHUNK // _L, pair_body, 0, unroll=False)
        d1 = pltpu.async_copy(rel_hbm.at[rp_v], r_v, sem)
        d2 = pltpu.async_copy(nrm_hbm.at[rp_v], n_v, sem)
        d3 = pltpu.async_copy(h2_hbm.at[pl.ds(base, CHUNK)], h_v, sem)
        d4 = pltpu.async_copy(t2_hbm.at[pl.ds(base, CHUNK)], t_v, sem)
        d1.wait()
        d2.wait()
        d3.wait()
        d4.wait()

        lanes = lax.iota(jnp.int32, _L)

        def row_body(i, svec):
            lane = lax.rem(i, _L)
            grp = i - lane
            lanevec = jnp.full((_L, 1), lane, jnp.int32)
            gs = pl.ds(grp, _L)
            off_h = _shuffle((hidx_v[gs] & 1) * D, lanevec)
            off_r = _shuffle((ridx_v[gs] & 1) * D, lanevec)
            off_t = _shuffle((tidx_v[gs] & 1) * D, lanevec)
            row = jnp.full((_L,), i, jnp.int32)

            def grab(buf, off, k):
                cols = lanes + off + (k * _L)
                return plsc.load_gather(buf, [row, cols])

            h = [grab(h_v, off_h, k) for k in range(_ND)]
            u = [grab(n_v, off_r, k) for k in range(_ND)]
            t = [grab(t_v, off_t, k) for k in range(_ND)]
            r = [grab(r_v, off_r, k) for k in range(_ND)]

            uu = _sum16(sum(u[k] * u[k] for k in range(_ND)))
            hu = _sum16(sum(h[k] * u[k] for k in range(_ND)))
            tu = _sum16(sum(t[k] * u[k] for k in range(_ND)))
            inv_den = jnp.float32(1.0) / jnp.maximum(uu, jnp.float32(1e-24))
            ah = hu * inv_den
            at = tu * inv_den
            hp = [h[k] - ah * u[k] for k in range(_ND)]
            tp = [t[k] - at * u[k] for k in range(_ND)]

            hh = _sum16(sum(hp[k] * hp[k] for k in range(_ND)))
            rr = _sum16(sum(r[k] * r[k] for k in range(_ND)))
            tt = _sum16(sum(tp[k] * tp[k] for k in range(_ND)))
            ih = _rsqrt_guard(hh)
            ir = _rsqrt_guard(rr)
            it = _rsqrt_guard(tt)

            sc = _sum16(sum(
                jnp.abs(hp[k] * ih + r[k] * ir - tp[k] * it)
                for k in range(_ND)))
            svec = jnp.where(lanes == lane, sc, svec)

            @pl.when(lane == _L - 1)
            def _():
                out_v[pl.ds(i - (_L - 1), _L)] = svec

            return svec

        lax.fori_loop(0, CHUNK, row_body, jnp.zeros((_L,), jnp.float32),
                      unroll=False)
        pltpu.sync_copy(out_v, out_hbm.at[pl.ds(base, CHUNK)])
        return carry

    lax.fori_loop(0, NCHUNK, chunk_body, 0, unroll=False)


@jax.jit
def _transh_sc(h_idx, r_idx, t_idx, ent2, rel2, nrm2):
    mesh = plsc.VectorSubcoreMesh(core_axis_name="c", subcore_axis_name="s", num_cores=1)
    h2, t2 = pl.kernel(
        _stage_body,
        out_type=(jax.ShapeDtypeStruct((B, 2 * D), jnp.float32),
                  jax.ShapeDtypeStruct((B, 2 * D), jnp.float32)),
        mesh=mesh,
        scratch_types=[
            pltpu.VMEM((CHUNK,), jnp.int32),
            pltpu.VMEM((CHUNK,), jnp.int32),
            pltpu.VMEM((CHUNK,), jnp.int32),
            pltpu.VMEM((CHUNK,), jnp.int32),
            pltpu.VMEM((CHUNK, 2 * D), jnp.float32),
            pltpu.VMEM((CHUNK, 2 * D), jnp.float32),
            pltpu.SemaphoreType.DMA,
        ],
        compiler_params=pltpu.CompilerParams(needs_layout_passes=False, skip_device_barrier=True),
    )(h_idx, t_idx, ent2)
    return pl.kernel(
        _math_body,
        out_type=jax.ShapeDtypeStruct((B,), jnp.float32),
        mesh=mesh,
        scratch_types=[
            pltpu.VMEM((CHUNK,), jnp.int32),
            pltpu.VMEM((CHUNK,), jnp.int32),
            pltpu.VMEM((CHUNK,), jnp.int32),
            pltpu.VMEM((CHUNK,), jnp.int32),
            pltpu.VMEM((CHUNK, 2 * D), jnp.float32),
            pltpu.VMEM((CHUNK, 2 * D), jnp.float32),
            pltpu.VMEM((CHUNK, 2 * D), jnp.float32),
            pltpu.VMEM((CHUNK, 2 * D), jnp.float32),
            pltpu.VMEM((CHUNK,), jnp.float32),
            pltpu.SemaphoreType.DMA,
        ],
        compiler_params=pltpu.CompilerParams(needs_layout_passes=False, skip_device_barrier=True),
    )(h_idx, r_idx, t_idx, rel2, nrm2, h2, t2)


def kernel(triplet_idx, entity_emb, relation_emb, norm_vec):
    cols = triplet_idx.T
    h_idx = cols[0]
    r_idx = cols[1]
    t_idx = cols[2]
    ent2 = entity_emb.reshape(500000, 2 * D)
    rel2 = relation_emb.reshape(500000, 2 * D)
    nrm2 = norm_vec.reshape(500000, 2 * D)
    return _transh_sc(h_idx, r_idx, t_idx, ent2, rel2, nrm2)


# speedup vs baseline: 1.1906x; 1.1906x over previous
import functools

import jax
import jax.numpy as jnp
from jax import lax
from jax.experimental import pallas as pl
from jax.experimental.pallas import tpu as pltpu
from jax.experimental.pallas import tpu_sc as plsc

B = 16384
D = 64
NC = 2
NS = 16
NW = NC * NS
ROWS_PER_W = B // NW
RING = 2

_L = 16
_ND = D // _L
_NG = ROWS_PER_W // _L

_DNUMS = lax.GatherDimensionNumbers(
    offset_dims=(), collapsed_slice_dims=(0,), start_index_map=(0,))


def _shuffle(v, perm):
    return lax.gather(v, perm, _DNUMS, slice_sizes=(1,),
                      mode=lax.GatherScatterMode.PROMISE_IN_BOUNDS)


def _sum16(v):
    lanes = lax.iota(jnp.int32, _L)
    for k in (1, 2, 4, 8):
        perm = jnp.reshape(lanes ^ k, (_L, 1))
        v = v + _shuffle(v, perm)
    return v


def _rsqrt_guard(s):
    i = lax.bitcast_convert_type(s, jnp.int32)
    i = jnp.int32(0x5F3759DF) - lax.shift_right_logical(i, 1)
    y = lax.bitcast_convert_type(i, jnp.float32)
    half = s * jnp.float32(0.5)
    for _ in range(3):
        y = y * (jnp.float32(1.5) - half * y * y)
    return jnp.minimum(y, jnp.float32(1e12))


def _lane(v, j):
    return jnp.squeeze(lax.slice(v, (j,), (j + 1,)))


def _body(h_idx_hbm, r_idx_hbm, t_idx_hbm, ent_hbm, rel_hbm, nrm_hbm,
          out_hbm, hidx_v, ridx_v, tidx_v, h_v, r_v, t_v, n_v, out_v, sem):
    wid = lax.axis_index("s") * NC + lax.axis_index("c")
    base = wid * ROWS_PER_W
    pltpu.sync_copy(h_idx_hbm.at[pl.ds(base, ROWS_PER_W)], hidx_v)
    pltpu.sync_copy(r_idx_hbm.at[pl.ds(base, ROWS_PER_W)], ridx_v)
    pltpu.sync_copy(t_idx_hbm.at[pl.ds(base, ROWS_PER_W)], tidx_v)

    lanes = lax.iota(jnp.int32, _L)

    def fetch(bh, br, bt, j, slot):
        oh = pl.multiple_of(_lane(bh, j), 8)
        orr = pl.multiple_of(_lane(br, j), 8)
        ot = pl.multiple_of(_lane(bt, j), 8)
        d1 = pltpu.async_copy(ent_hbm.at[pl.ds(oh, 8), :], h_v.at[slot], sem)
        d2 = pltpu.async_copy(rel_hbm.at[pl.ds(orr, 8), :], r_v.at[slot], sem)
        d3 = pltpu.async_copy(ent_hbm.at[pl.ds(ot, 8), :], t_v.at[slot], sem)
        d4 = pltpu.async_copy(nrm_hbm.at[pl.ds(orr, 8), :], n_v.at[slot], sem)
        return (d1, d2, d3, d4)

    def drain(descs):
        for d in descs:
            d.wait()

    def grp_body(g, carry):
        gs = pl.ds(g * _L, _L)
        ivh = hidx_v[gs]
        ivr = ridx_v[gs]
        ivt = tidx_v[gs]
        bh = lax.shift_right_logical(ivh, 3) * 8
        br = lax.shift_right_logical(ivr, 3) * 8
        bt = lax.shift_right_logical(ivt, 3) * 8
        sh = ivh & 7
        sr = ivr & 7
        st = ivt & 7

        svec = jnp.zeros((_L,), jnp.float32)
        pend = fetch(bh, br, bt, 0, 0)
        for j in range(_L):
            drain(pend)
            if j + 1 < _L:
                pend = fetch(bh, br, bt, j + 1, (j + 1) % RING)
            slot = j % RING
            jh = _lane(sh, j)
            jr = _lane(sr, j)
            jt = _lane(st, j)
            h = [h_v[slot, jh, pl.ds(k * _L, _L)] for k in range(_ND)]
            u = [n_v[slot, jr, pl.ds(k * _L, _L)] for k in range(_ND)]
            t = [t_v[slot, jt, pl.ds(k * _L, _L)] for k in range(_ND)]
            r = [r_v[slot, jr, pl.ds(k * _L, _L)] for k in range(_ND)]

            uu = _sum16(sum(u[k] * u[k] for k in range(_ND)))
            hu = _sum16(sum(h[k] * u[k] for k in range(_ND)))
            tu = _sum16(sum(t[k] * u[k] for k in range(_ND)))
            inv_den = jnp.float32(1.0) / jnp.maximum(uu, jnp.float32(1e-24))
            ah = hu * inv_den
            at = tu * inv_den
            hp = [h[k] - ah * u[k] for k in range(_ND)]
            tp = [t[k] - at * u[k] for k in range(_ND)]

            hh = _sum16(sum(hp[k] * hp[k] for k in range(_ND)))
            rr = _sum16(sum(r[k] * r[k] for k in range(_ND)))
            tt = _sum16(sum(tp[k] * tp[k] for k in range(_ND)))
            ih = _rsqrt_guard(hh)
            ir = _rsqrt_guard(rr)
            it = _rsqrt_guard(tt)

            sc = _sum16(sum(
                jnp.abs(hp[k] * ih + r[k] * ir - tp[k] * it)
                for k in range(_ND)))
            svec = jnp.where(lanes == j, sc, svec)

        out_v[gs] = svec
        return carry

    lax.fori_loop(0, _NG, grp_body, 0, unroll=False)
    pltpu.sync_copy(out_v, out_hbm.at[pl.ds(base, ROWS_PER_W)])


@jax.jit
def _transh_sc(h_idx, r_idx, t_idx, ent, rel, nrm):
    mesh = plsc.VectorSubcoreMesh(core_axis_name="c", subcore_axis_name="s")
    return pl.kernel(
        _body,
        out_type=jax.ShapeDtypeStruct((B,), jnp.float32),
        mesh=mesh,
        scratch_types=[
            pltpu.VMEM((ROWS_PER_W,), jnp.int32),
            pltpu.VMEM((ROWS_PER_W,), jnp.int32),
            pltpu.VMEM((ROWS_PER_W,), jnp.int32),
            pltpu.VMEM((RING, 8, D), jnp.float32),
            pltpu.VMEM((RING, 8, D), jnp.float32),
            pltpu.VMEM((RING, 8, D), jnp.float32),
            pltpu.VMEM((RING, 8, D), jnp.float32),
            pltpu.VMEM((ROWS_PER_W,), jnp.float32),
            pltpu.SemaphoreType.DMA,
        ],
        compiler_params=pltpu.CompilerParams(needs_layout_passes=False),
    )(h_idx, r_idx, t_idx, ent, rel, nrm)


def kernel(triplet_idx, entity_emb, relation_emb, norm_vec):
    cols = triplet_idx.T
    h_idx = cols[0]
    r_idx = cols[1]
    t_idx = cols[2]
    return _transh_sc(h_idx, r_idx, t_idx, entity_emb, relation_emb, norm_vec)


# speedup vs baseline: 1.4392x; 1.2088x over previous
import functools

import jax
import jax.numpy as jnp
from jax import lax
from jax.experimental import pallas as pl
from jax.experimental.pallas import tpu as pltpu
from jax.experimental.pallas import tpu_sc as plsc

B = 16384
D = 64
NC = 2
NS = 16
NW = NC * NS
ROWS_PER_W = B // NW
RING = 4
DEPTH = 3

_L = 16
_ND = D // _L
_NG = ROWS_PER_W // _L

_DNUMS = lax.GatherDimensionNumbers(
    offset_dims=(), collapsed_slice_dims=(0,), start_index_map=(0,))


def _shuffle(v, perm):
    return lax.gather(v, perm, _DNUMS, slice_sizes=(1,),
                      mode=lax.GatherScatterMode.PROMISE_IN_BOUNDS)


def _sum16(v):
    lanes = lax.iota(jnp.int32, _L)
    for k in (1, 2, 4, 8):
        perm = jnp.reshape(lanes ^ k, (_L, 1))
        v = v + _shuffle(v, perm)
    return v


def _rsqrt_guard(s):
    i = lax.bitcast_convert_type(s, jnp.int32)
    i = jnp.int32(0x5F3759DF) - lax.shift_right_logical(i, 1)
    y = lax.bitcast_convert_type(i, jnp.float32)
    half = s * jnp.float32(0.5)
    for _ in range(3):
        y = y * (jnp.float32(1.5) - half * y * y)
    return jnp.minimum(y, jnp.float32(1e12))


def _lane(v, j):
    return jnp.squeeze(lax.slice(v, (j,), (j + 1,)))


def _body(h_idx_hbm, r_idx_hbm, t_idx_hbm, ent_hbm, rel_hbm, nrm_hbm,
          out_hbm, hidx_v, ridx_v, tidx_v, h_v, r_v, t_v, n_v, out_v, sem):
    wid = lax.axis_index("s") * NC + lax.axis_index("c")
    base = wid * ROWS_PER_W
    pltpu.sync_copy(h_idx_hbm.at[pl.ds(base, ROWS_PER_W)], hidx_v)
    pltpu.sync_copy(r_idx_hbm.at[pl.ds(base, ROWS_PER_W)], ridx_v)
    pltpu.sync_copy(t_idx_hbm.at[pl.ds(base, ROWS_PER_W)], tidx_v)

    lanes = lax.iota(jnp.int32, _L)

    def fetch(bh, br, bt, j, slot):
        oh = pl.multiple_of(_lane(bh, j), 8)
        orr = pl.multiple_of(_lane(br, j), 8)
        ot = pl.multiple_of(_lane(bt, j), 8)
        d1 = pltpu.async_copy(ent_hbm.at[pl.ds(oh, 8), :], h_v.at[slot], sem)
        d2 = pltpu.async_copy(rel_hbm.at[pl.ds(orr, 8), :], r_v.at[slot], sem)
        d3 = pltpu.async_copy(ent_hbm.at[pl.ds(ot, 8), :], t_v.at[slot], sem)
        d4 = pltpu.async_copy(nrm_hbm.at[pl.ds(orr, 8), :], n_v.at[slot], sem)
        return (d1, d2, d3, d4)

    def drain(descs):
        for d in descs:
            d.wait()

    def grp_body(g, carry):
        gs = pl.ds(g * _L, _L)
        ivh = hidx_v[gs]
        ivr = ridx_v[gs]
        ivt = tidx_v[gs]
        bh = lax.shift_right_logical(ivh, 3) * 8
        br = lax.shift_right_logical(ivr, 3) * 8
        bt = lax.shift_right_logical(ivt, 3) * 8
        sh = ivh & 7
        sr = ivr & 7
        st = ivt & 7

        svec = jnp.zeros((_L,), jnp.float32)
        pend = [fetch(bh, br, bt, jj, jj % RING) for jj in range(DEPTH)]
        for j in range(_L):
            drain(pend.pop(0))
            if j + DEPTH < _L:
                pend.append(fetch(bh, br, bt, j + DEPTH, (j + DEPTH) % RING))
            slot = j % RING
            jh = _lane(sh, j)
            jr = _lane(sr, j)
            jt = _lane(st, j)
            h = [h_v[slot, jh, pl.ds(k * _L, _L)] for k in range(_ND)]
            u = [n_v[slot, jr, pl.ds(k * _L, _L)] for k in range(_ND)]
            t = [t_v[slot, jt, pl.ds(k * _L, _L)] for k in range(_ND)]
            r = [r_v[slot, jr, pl.ds(k * _L, _L)] for k in range(_ND)]

            uu = _sum16(sum(u[k] * u[k] for k in range(_ND)))
            hu = _sum16(sum(h[k] * u[k] for k in range(_ND)))
            tu = _sum16(sum(t[k] * u[k] for k in range(_ND)))
            inv_den = jnp.float32(1.0) / jnp.maximum(uu, jnp.float32(1e-24))
            ah = hu * inv_den
            at = tu * inv_den
            hp = [h[k] - ah * u[k] for k in range(_ND)]
            tp = [t[k] - at * u[k] for k in range(_ND)]

            hh = _sum16(sum(hp[k] * hp[k] for k in range(_ND)))
            rr = _sum16(sum(r[k] * r[k] for k in range(_ND)))
            tt = _sum16(sum(tp[k] * tp[k] for k in range(_ND)))
            ih = _rsqrt_guard(hh)
            ir = _rsqrt_guard(rr)
            it = _rsqrt_guard(tt)

            sc = _sum16(sum(
                jnp.abs(hp[k] * ih + r[k] * ir - tp[k] * it)
                for k in range(_ND)))
            svec = jnp.where(lanes == j, sc, svec)

        out_v[gs] = svec
        return carry

    lax.fori_loop(0, _NG, grp_body, 0, unroll=False)
    pltpu.sync_copy(out_v, out_hbm.at[pl.ds(base, ROWS_PER_W)])


@jax.jit
def _transh_sc(h_idx, r_idx, t_idx, ent, rel, nrm):
    mesh = plsc.VectorSubcoreMesh(core_axis_name="c", subcore_axis_name="s")
    return pl.kernel(
        _body,
        out_type=jax.ShapeDtypeStruct((B,), jnp.float32),
        mesh=mesh,
        scratch_types=[
            pltpu.VMEM((ROWS_PER_W,), jnp.int32),
            pltpu.VMEM((ROWS_PER_W,), jnp.int32),
            pltpu.VMEM((ROWS_PER_W,), jnp.int32),
            pltpu.VMEM((RING, 8, D), jnp.float32),
            pltpu.VMEM((RING, 8, D), jnp.float32),
            pltpu.VMEM((RING, 8, D), jnp.float32),
            pltpu.VMEM((RING, 8, D), jnp.float32),
            pltpu.VMEM((ROWS_PER_W,), jnp.float32),
            pltpu.SemaphoreType.DMA,
        ],
        compiler_params=pltpu.CompilerParams(needs_layout_passes=False),
    )(h_idx, r_idx, t_idx, ent, rel, nrm)


def kernel(triplet_idx, entity_emb, relation_emb, norm_vec):
    cols = triplet_idx.T
    h_idx = cols[0]
    r_idx = cols[1]
    t_idx = cols[2]
    return _transh_sc(h_idx, r_idx, t_idx, entity_emb, relation_emb, norm_vec)


# speedup vs baseline: 1.4800x; 1.0283x over previous
import functools

import jax
import jax.numpy as jnp
from jax import lax
from jax.experimental import pallas as pl
from jax.experimental.pallas import tpu as pltpu
from jax.experimental.pallas import tpu_sc as plsc

B = 16384
D = 64
NC = 2
NS = 16
NW = NC * NS
ROWS_PER_W = B // NW
RING = 8
DEPTH = 6

_L = 16
_ND = D // _L
_NG = ROWS_PER_W // _L

_DNUMS = lax.GatherDimensionNumbers(
    offset_dims=(), collapsed_slice_dims=(0,), start_index_map=(0,))


def _shuffle(v, perm):
    return lax.gather(v, perm, _DNUMS, slice_sizes=(1,),
                      mode=lax.GatherScatterMode.PROMISE_IN_BOUNDS)


def _sum16(v):
    lanes = lax.iota(jnp.int32, _L)
    for k in (1, 2, 4, 8):
        perm = jnp.reshape(lanes ^ k, (_L, 1))
        v = v + _shuffle(v, perm)
    return v


def _rsqrt_guard(s):
    i = lax.bitcast_convert_type(s, jnp.int32)
    i = jnp.int32(0x5F3759DF) - lax.shift_right_logical(i, 1)
    y = lax.bitcast_convert_type(i, jnp.float32)
    half = s * jnp.float32(0.5)
    for _ in range(3):
        y = y * (jnp.float32(1.5) - half * y * y)
    return jnp.minimum(y, jnp.float32(1e12))


def _lane(v, j):
    return jnp.squeeze(lax.slice(v, (j,), (j + 1,)))


def _body(h_idx_hbm, r_idx_hbm, t_idx_hbm, ent_hbm, rel_hbm, nrm_hbm,
          out_hbm, hidx_v, ridx_v, tidx_v, h_v, r_v, t_v, n_v, out_v, sem):
    wid = lax.axis_index("s") * NC + lax.axis_index("c")
    base = wid * ROWS_PER_W
    pltpu.sync_copy(h_idx_hbm.at[pl.ds(base, ROWS_PER_W)], hidx_v)
    pltpu.sync_copy(r_idx_hbm.at[pl.ds(base, ROWS_PER_W)], ridx_v)
    pltpu.sync_copy(t_idx_hbm.at[pl.ds(base, ROWS_PER_W)], tidx_v)

    lanes = lax.iota(jnp.int32, _L)

    def fetch(bh, br, bt, j, slot):
        oh = pl.multiple_of(_lane(bh, j), 8)
        orr = pl.multiple_of(_lane(br, j), 8)
        ot = pl.multiple_of(_lane(bt, j), 8)
        d1 = pltpu.async_copy(ent_hbm.at[pl.ds(oh, 8), :], h_v.at[slot], sem)
        d2 = pltpu.async_copy(rel_hbm.at[pl.ds(orr, 8), :], r_v.at[slot], sem)
        d3 = pltpu.async_copy(ent_hbm.at[pl.ds(ot, 8), :], t_v.at[slot], sem)
        d4 = pltpu.async_copy(nrm_hbm.at[pl.ds(orr, 8), :], n_v.at[slot], sem)
        return (d1, d2, d3, d4)

    def drain(descs):
        for d in descs:
            d.wait()

    def grp_body(g, carry):
        gs = pl.ds(g * _L, _L)
        ivh = hidx_v[gs]
        ivr = ridx_v[gs]
        ivt = tidx_v[gs]
        bh = lax.shift_right_logical(ivh, 3) * 8
        br = lax.shift_right_logical(ivr, 3) * 8
        bt = lax.shift_right_logical(ivt, 3) * 8
        sh = ivh & 7
        sr = ivr & 7
        st = ivt & 7

        svec = jnp.zeros((_L,), jnp.float32)
        pend = [fetch(bh, br, bt, jj, jj % RING) for jj in range(DEPTH)]
        for j in range(_L):
            drain(pend.pop(0))
            if j + DEPTH < _L:
                pend.append(fetch(bh, br, bt, j + DEPTH, (j + DEPTH) % RING))
            slot = j % RING
            jh = _lane(sh, j)
            jr = _lane(sr, j)
            jt = _lane(st, j)
            h = [h_v[slot, jh, pl.ds(k * _L, _L)] for k in range(_ND)]
            u = [n_v[slot, jr, pl.ds(k * _L, _L)] for k in range(_ND)]
            t = [t_v[slot, jt, pl.ds(k * _L, _L)] for k in range(_ND)]
            r = [r_v[slot, jr, pl.ds(k * _L, _L)] for k in range(_ND)]

            uu = _sum16(sum(u[k] * u[k] for k in range(_ND)))
            hu = _sum16(sum(h[k] * u[k] for k in range(_ND)))
            tu = _sum16(sum(t[k] * u[k] for k in range(_ND)))
            inv_den = jnp.float32(1.0) / jnp.maximum(uu, jnp.float32(1e-24))
            ah = hu * inv_den
            at = tu * inv_den
            hp = [h[k] - ah * u[k] for k in range(_ND)]
            tp = [t[k] - at * u[k] for k in range(_ND)]

            hh = _sum16(sum(hp[k] * hp[k] for k in range(_ND)))
            rr = _sum16(sum(r[k] * r[k] for k in range(_ND)))
            tt = _sum16(sum(tp[k] * tp[k] for k in range(_ND)))
            ih = _rsqrt_guard(hh)
            ir = _rsqrt_guard(rr)
            it = _rsqrt_guard(tt)

            sc = _sum16(sum(
                jnp.abs(hp[k] * ih + r[k] * ir - tp[k] * it)
                for k in range(_ND)))
            svec = jnp.where(lanes == j, sc, svec)

        out_v[gs] = svec
        return carry

    lax.fori_loop(0, _NG, grp_body, 0, unroll=False)
    pltpu.sync_copy(out_v, out_hbm.at[pl.ds(base, ROWS_PER_W)])


@jax.jit
def _transh_sc(h_idx, r_idx, t_idx, ent, rel, nrm):
    mesh = plsc.VectorSubcoreMesh(core_axis_name="c", subcore_axis_name="s")
    return pl.kernel(
        _body,
        out_type=jax.ShapeDtypeStruct((B,), jnp.float32),
        mesh=mesh,
        scratch_types=[
            pltpu.VMEM((ROWS_PER_W,), jnp.int32),
            pltpu.VMEM((ROWS_PER_W,), jnp.int32),
            pltpu.VMEM((ROWS_PER_W,), jnp.int32),
            pltpu.VMEM((RING, 8, D), jnp.float32),
            pltpu.VMEM((RING, 8, D), jnp.float32),
            pltpu.VMEM((RING, 8, D), jnp.float32),
            pltpu.VMEM((RING, 8, D), jnp.float32),
            pltpu.VMEM((ROWS_PER_W,), jnp.float32),
            pltpu.SemaphoreType.DMA,
        ],
        compiler_params=pltpu.CompilerParams(needs_layout_passes=False),
    )(h_idx, r_idx, t_idx, ent, rel, nrm)


def kernel(triplet_idx, entity_emb, relation_emb, norm_vec):
    cols = triplet_idx.T
    h_idx = cols[0]
    r_idx = cols[1]
    t_idx = cols[2]
    return _transh_sc(h_idx, r_idx, t_idx, entity_emb, relation_emb, norm_vec)


# speedup vs baseline: 1.4926x; 1.0085x over previous
import functools

import jax
import jax.numpy as jnp
from jax import lax
from jax.experimental import pallas as pl
from jax.experimental.pallas import tpu as pltpu
from jax.experimental.pallas import tpu_sc as plsc

B = 16384
D = 64
NC = 2
NS = 16
NW = NC * NS
ROWS_PER_W = B // NW
RING = 8
DEPTH = 6

_L = 16
_ND = D // _L
_NG = ROWS_PER_W // _L

_DNUMS = lax.GatherDimensionNumbers(
    offset_dims=(), collapsed_slice_dims=(0,), start_index_map=(0,))


def _shuffle(v, perm):
    return lax.gather(v, perm, _DNUMS, slice_sizes=(1,),
                      mode=lax.GatherScatterMode.PROMISE_IN_BOUNDS)


def _sum16(v):
    lanes = lax.iota(jnp.int32, _L)
    for k in (1, 2, 4, 8):
        perm = jnp.reshape(lanes ^ k, (_L, 1))
        v = v + _shuffle(v, perm)
    return v


def _rsqrt_guard(s):
    i = lax.bitcast_convert_type(s, jnp.int32)
    i = jnp.int32(0x5F3759DF) - lax.shift_right_logical(i, 1)
    y = lax.bitcast_convert_type(i, jnp.float32)
    half = s * jnp.float32(0.5)
    for _ in range(3):
        y = y * (jnp.float32(1.5) - half * y * y)
    return jnp.minimum(y, jnp.float32(1e12))


def _lane(v, j):
    return jnp.squeeze(lax.slice(v, (j,), (j + 1,)))


def _body(h_idx_hbm, r_idx_hbm, t_idx_hbm, ent_hbm, rel_hbm, nrm_hbm,
          out_hbm, hidx_v, ridx_v, tidx_v, h_v, r_v, t_v, n_v, out_v,
          drain_v, sem):
    wid = lax.axis_index("s") * NC + lax.axis_index("c")
    base = wid * ROWS_PER_W
    pltpu.sync_copy(h_idx_hbm.at[pl.ds(base, ROWS_PER_W)], hidx_v)
    pltpu.sync_copy(r_idx_hbm.at[pl.ds(base, ROWS_PER_W)], ridx_v)
    pltpu.sync_copy(t_idx_hbm.at[pl.ds(base, ROWS_PER_W)], tidx_v)

    lanes = lax.iota(jnp.int32, _L)

    def fetch(bh, br, bt, j, slot):
        oh = pl.multiple_of(_lane(bh, j), 8)
        orr = pl.multiple_of(_lane(br, j), 8)
        ot = pl.multiple_of(_lane(bt, j), 8)
        pltpu.async_copy(ent_hbm.at[pl.ds(oh, 8), :], h_v.at[slot], sem)
        pltpu.async_copy(rel_hbm.at[pl.ds(orr, 8), :], r_v.at[slot], sem)
        pltpu.async_copy(ent_hbm.at[pl.ds(ot, 8), :], t_v.at[slot], sem)
        pltpu.async_copy(nrm_hbm.at[pl.ds(orr, 8), :], n_v.at[slot], sem)
        return slot

    def drain_row():
        pltpu.make_async_copy(
            ent_hbm.at[pl.ds(0, 32), :], drain_v, sem).wait()

    def grp_body(g, carry):
        gs = pl.ds(g * _L, _L)
        ivh = hidx_v[gs]
        ivr = ridx_v[gs]
        ivt = tidx_v[gs]
        bh = lax.shift_right_logical(ivh, 3) * 8
        br = lax.shift_right_logical(ivr, 3) * 8
        bt = lax.shift_right_logical(ivt, 3) * 8
        sh = ivh & 7
        sr = ivr & 7
        st = ivt & 7

        svec = jnp.zeros((_L,), jnp.float32)
        for jj in range(DEPTH):
            fetch(bh, br, bt, jj, jj % RING)
        for j in range(_L):
            drain_row()
            if j + DEPTH < _L:
                fetch(bh, br, bt, j + DEPTH, (j + DEPTH) % RING)
            slot = j % RING
            jh = _lane(sh, j)
            jr = _lane(sr, j)
            jt = _lane(st, j)
            h = [h_v[slot, jh, pl.ds(k * _L, _L)] for k in range(_ND)]
            u = [n_v[slot, jr, pl.ds(k * _L, _L)] for k in range(_ND)]
            t = [t_v[slot, jt, pl.ds(k * _L, _L)] for k in range(_ND)]
            r = [r_v[slot, jr, pl.ds(k * _L, _L)] for k in range(_ND)]

            uu = _sum16(sum(u[k] * u[k] for k in range(_ND)))
            hu = _sum16(sum(h[k] * u[k] for k in range(_ND)))
            tu = _sum16(sum(t[k] * u[k] for k in range(_ND)))
            inv_den = jnp.float32(1.0) / jnp.maximum(uu, jnp.float32(1e-24))
            ah = hu * inv_den
            at = tu * inv_den
            hp = [h[k] - ah * u[k] for k in range(_ND)]
            tp = [t[k] - at * u[k] for k in range(_ND)]

            hh = _sum16(sum(hp[k] * hp[k] for k in range(_ND)))
            rr = _sum16(sum(r[k] * r[k] for k in range(_ND)))
            tt = _sum16(sum(tp[k] * tp[k] for k in range(_ND)))
            ih = _rsqrt_guard(hh)
            ir = _rsqrt_guard(rr)
            it = _rsqrt_guard(tt)

            sc = _sum16(sum(
                jnp.abs(hp[k] * ih + r[k] * ir - tp[k] * it)
                for k in range(_ND)))
            svec = jnp.where(lanes == j, sc, svec)

        out_v[gs] = svec
        return carry

    lax.fori_loop(0, _NG, grp_body, 0, unroll=False)
    pltpu.sync_copy(out_v, out_hbm.at[pl.ds(base, ROWS_PER_W)])


@jax.jit
def _transh_sc(h_idx, r_idx, t_idx, ent, rel, nrm):
    mesh = plsc.VectorSubcoreMesh(core_axis_name="c", subcore_axis_name="s")
    return pl.kernel(
        _body,
        out_type=jax.ShapeDtypeStruct((B,), jnp.float32),
        mesh=mesh,
        scratch_types=[
            pltpu.VMEM((ROWS_PER_W,), jnp.int32),
            pltpu.VMEM((ROWS_PER_W,), jnp.int32),
            pltpu.VMEM((ROWS_PER_W,), jnp.int32),
            pltpu.VMEM((RING, 8, D), jnp.float32),
            pltpu.VMEM((RING, 8, D), jnp.float32),
            pltpu.VMEM((RING, 8, D), jnp.float32),
            pltpu.VMEM((RING, 8, D), jnp.float32),
            pltpu.VMEM((ROWS_PER_W,), jnp.float32),
            pltpu.VMEM((32, D), jnp.float32),
            pltpu.SemaphoreType.DMA,
        ],
        compiler_params=pltpu.CompilerParams(needs_layout_passes=False),
    )(h_idx, r_idx, t_idx, ent, rel, nrm)


def kernel(triplet_idx, entity_emb, relation_emb, norm_vec):
    cols = triplet_idx.T
    h_idx = cols[0]
    r_idx = cols[1]
    t_idx = cols[2]
    return _transh_sc(h_idx, r_idx, t_idx, entity_emb, relation_emb, norm_vec)


# speedup vs baseline: 1.5024x; 1.0066x over previous
import functools

import jax
import jax.numpy as jnp
from jax import lax
from jax.experimental import pallas as pl
from jax.experimental.pallas import tpu as pltpu
from jax.experimental.pallas import tpu_sc as plsc

B = 16384
D = 64
NC = 2
NS = 16
NW = NC * NS
ROWS_PER_W = B // NW
RING = 16

_L = 16
_ND = D // _L
_NG = ROWS_PER_W // _L

_DNUMS = lax.GatherDimensionNumbers(
    offset_dims=(), collapsed_slice_dims=(0,), start_index_map=(0,))


def _shuffle(v, perm):
    return lax.gather(v, perm, _DNUMS, slice_sizes=(1,),
                      mode=lax.GatherScatterMode.PROMISE_IN_BOUNDS)


def _sum16(v):
    lanes = lax.iota(jnp.int32, _L)
    for k in (1, 2, 4, 8):
        perm = jnp.reshape(lanes ^ k, (_L, 1))
        v = v + _shuffle(v, perm)
    return v


def _rsqrt_guard(s):
    i = lax.bitcast_convert_type(s, jnp.int32)
    i = jnp.int32(0x5F3759DF) - lax.shift_right_logical(i, 1)
    y = lax.bitcast_convert_type(i, jnp.float32)
    half = s * jnp.float32(0.5)
    for _ in range(3):
        y = y * (jnp.float32(1.5) - half * y * y)
    return jnp.minimum(y, jnp.float32(1e12))


def _lane(v, j):
    return jnp.squeeze(lax.slice(v, (j,), (j + 1,)))


def _body(h_idx_hbm, r_idx_hbm, t_idx_hbm, ent_hbm, rel_hbm, nrm_hbm,
          out_hbm, hidx_v, ridx_v, tidx_v, h_v, r_v, t_v, n_v, out_v,
          drain_v, sem):
    wid = lax.axis_index("s") * NC + lax.axis_index("c")
    base = wid * ROWS_PER_W
    pltpu.sync_copy(h_idx_hbm.at[pl.ds(base, ROWS_PER_W)], hidx_v)
    pltpu.sync_copy(r_idx_hbm.at[pl.ds(base, ROWS_PER_W)], ridx_v)
    pltpu.sync_copy(t_idx_hbm.at[pl.ds(base, ROWS_PER_W)], tidx_v)

    lanes = lax.iota(jnp.int32, _L)

    def fetch(bh, br, bt, j, slot):
        oh = pl.multiple_of(_lane(bh, j), 8)
        orr = pl.multiple_of(_lane(br, j), 8)
        ot = pl.multiple_of(_lane(bt, j), 8)
        pltpu.async_copy(ent_hbm.at[pl.ds(oh, 8), :], h_v.at[slot], sem)
        pltpu.async_copy(rel_hbm.at[pl.ds(orr, 8), :], r_v.at[slot], sem)
        pltpu.async_copy(ent_hbm.at[pl.ds(ot, 8), :], t_v.at[slot], sem)
        pltpu.async_copy(nrm_hbm.at[pl.ds(orr, 8), :], n_v.at[slot], sem)
        return slot

    def drain_row():
        pltpu.make_async_copy(
            ent_hbm.at[pl.ds(0, 32), :], drain_v, sem).wait()

    def grp_body(g, carry):
        gs = pl.ds(g * _L, _L)
        ivh = hidx_v[gs]
        ivr = ridx_v[gs]
        ivt = tidx_v[gs]
        bh = lax.shift_right_logical(ivh, 3) * 8
        br = lax.shift_right_logical(ivr, 3) * 8
        bt = lax.shift_right_logical(ivt, 3) * 8
        sh = ivh & 7
        sr = ivr & 7
        st = ivt & 7

        for jj in range(_L):
            fetch(bh, br, bt, jj, jj)

        def row_body(j, svec):
            drain_row()
            slot = jnp.full((_L,), j, jnp.int32)
            lj = jnp.full((_L, 1), j, jnp.int32)
            sub_h = _shuffle(sh, lj)
            sub_r = _shuffle(sr, lj)
            sub_t = _shuffle(st, lj)

            def grab(buf, sub, k):
                cols = lanes + (k * _L)
                return plsc.load_gather(buf, [slot, sub, cols])

            h = [grab(h_v, sub_h, k) for k in range(_ND)]
            u = [grab(n_v, sub_r, k) for k in range(_ND)]
            t = [grab(t_v, sub_t, k) for k in range(_ND)]
            r = [grab(r_v, sub_r, k) for k in range(_ND)]

            uu = _sum16(sum(u[k] * u[k] for k in range(_ND)))
            hu = _sum16(sum(h[k] * u[k] for k in range(_ND)))
            tu = _sum16(sum(t[k] * u[k] for k in range(_ND)))
            inv_den = jnp.float32(1.0) / jnp.maximum(uu, jnp.float32(1e-24))
            ah = hu * inv_den
            at = tu * inv_den
            hp = [h[k] - ah * u[k] for k in range(_ND)]
            tp = [t[k] - at * u[k] for k in range(_ND)]

            hh = _sum16(sum(hp[k] * hp[k] for k in range(_ND)))
            rr = _sum16(sum(r[k] * r[k] for k in range(_ND)))
            tt = _sum16(sum(tp[k] * tp[k] for k in range(_ND)))
            ih = _rsqrt_guard(hh)
            ir = _rsqrt_guard(rr)
            it = _rsqrt_guard(tt)

            sc = _sum16(sum(
                jnp.abs(hp[k] * ih + r[k] * ir - tp[k] * it)
                for k in range(_ND)))
            return jnp.where(lanes == j, sc, svec)

        svec = lax.fori_loop(0, _L, row_body, jnp.zeros((_L,), jnp.float32),
                             unroll=False)
        out_v[gs] = svec
        return carry

    lax.fori_loop(0, _NG, grp_body, 0, unroll=False)
    pltpu.sync_copy(out_v, out_hbm.at[pl.ds(base, ROWS_PER_W)])


@jax.jit
def _transh_sc(h_idx, r_idx, t_idx, ent, rel, nrm):
    mesh = plsc.VectorSubcoreMesh(core_axis_name="c", subcore_axis_name="s")
    return pl.kernel(
        _body,
        out_type=jax.ShapeDtypeStruct((B,), jnp.float32),
        mesh=mesh,
        scratch_types=[
            pltpu.VMEM((ROWS_PER_W,), jnp.int32),
            pltpu.VMEM((ROWS_PER_W,), jnp.int32),
            pltpu.VMEM((ROWS_PER_W,), jnp.int32),
            pltpu.VMEM((RING, 8, D), jnp.float32),
            pltpu.VMEM((RING, 8, D), jnp.float32),
            pltpu.VMEM((RING, 8, D), jnp.float32),
            pltpu.VMEM((RING, 8, D), jnp.float32),
            pltpu.VMEM((ROWS_PER_W,), jnp.float32),
            pltpu.VMEM((32, D), jnp.float32),
            pltpu.SemaphoreType.DMA,
        ],
        compiler_params=pltpu.CompilerParams(needs_layout_passes=False),
    )(h_idx, r_idx, t_idx, ent, rel, nrm)


def kernel(triplet_idx, entity_emb, relation_emb, norm_vec):
    cols = triplet_idx.T
    h_idx = cols[0]
    r_idx = cols[1]
    t_idx = cols[2]
    return _transh_sc(h_idx, r_idx, t_idx, entity_emb, relation_emb, norm_vec)
